# Initial kernel scaffold; baseline (speedup 1.0000x reference)
import jax
import jax.numpy as jnp
from jax.experimental import pallas as pl


def kernel(x, expert_indices, w1, w2, w3):
    raise NotImplementedError("write your pallas kernel here")


# speedup vs baseline: 8.0838x; 8.0838x over previous
import jax
import jax.numpy as jnp
from jax.experimental import pallas as pl
from jax.experimental.pallas import tpu as pltpu


def _ffn_body(slots_ref, x_ref, sel_ref, w1_ref, w3_ref, w2_ref, out_ref):
    j = pl.program_id(0)
    e = pl.program_id(1)
    x = x_ref[...]
    w1t = w1_ref[0]
    w3t = w3_ref[0]
    w2t = w2_ref[0]

    dn = (((1,), (1,)), ((), ()))
    h1 = jax.lax.dot_general(x, w1t, dn, preferred_element_type=jnp.float32)
    h3 = jax.lax.dot_general(x, w3t, dn, preferred_element_type=jnp.float32)
    g = h1 * jax.lax.logistic(h1) * h3
    y = jax.lax.dot_general(g, w2t, dn, preferred_element_type=jnp.float32)

    @pl.when((j == 0) & (e == 0))
    def _():
        out_ref[...] = jnp.zeros_like(out_ref)

    sel = sel_ref[0]
    out_ref[...] += jax.lax.dot_general(
        sel, y, (((1,), (0,)), ((), ())), preferred_element_type=jnp.float32)


def kernel(x, expert_indices, w1, w2, w3):
    E, I, D = w1.shape
    T, K = expert_indices.shape
    TK = T * K
    IT = 1024
    J = I // IT

    idx_flat = expert_indices.astype(jnp.int32).reshape(-1)
    flat = jnp.sort(idx_flat)
    is_new = jnp.concatenate(
        [jnp.ones((1,), jnp.bool_), flat[1:] != flat[:-1]])
    pos = jnp.cumsum(is_new) - 1
    slots = jnp.full((E,), flat[-1], jnp.int32)
    slots = slots.at[pos].set(flat)
    valid = jnp.concatenate(
        [jnp.ones((1,), jnp.bool_), slots[1:] != slots[:-1]])
    assign = valid[:, None] & (idx_flat[None, :] == slots[:, None])
    onehot = (jnp.arange(TK)[:, None] // K
              == jnp.arange(T)[None, :]).astype(jnp.float32)
    sel = assign[:, :, None].astype(jnp.float32) * onehot[None]

    grid_spec = pltpu.PrefetchScalarGridSpec(
        num_scalar_prefetch=1,
        grid=(J, E),
        in_specs=[
            pl.BlockSpec((T, D), lambda j, e, slots: (0, 0)),
            pl.BlockSpec((1, TK, T), lambda j, e, slots: (e, 0, 0)),
            pl.BlockSpec((1, IT, D), lambda j, e, slots: (slots[e], j, 0)),
            pl.BlockSpec((1, IT, D), lambda j, e, slots: (slots[e], j, 0)),
            pl.BlockSpec((1, D, IT), lambda j, e, slots: (slots[e], 0, j)),
        ],
        out_specs=pl.BlockSpec((TK, D), lambda j, e, slots: (0, 0)),
    )

    out = pl.pallas_call(
        _ffn_body,
        grid_spec=grid_spec,
        out_shape=jax.ShapeDtypeStruct((TK, D), jnp.float32),
    )(slots, x, sel, w1, w3, w2)
    return out.reshape(T, K, D)


# speedup vs baseline: 8.2060x; 1.0151x over previous
import jax
import jax.numpy as jnp
from jax.experimental import pallas as pl
from jax.experimental.pallas import tpu as pltpu


def _ffn_body(slots_ref, x_ref, sel_ref, w1_ref, w3_ref, w2_ref, out_ref):
    j = pl.program_id(0)
    e = pl.program_id(1)
    x = x_ref[...]
    w1t = w1_ref[0]
    w3t = w3_ref[0]
    w2t = w2_ref[0]

    dn = (((1,), (1,)), ((), ()))
    h1 = jax.lax.dot_general(x, w1t, dn, preferred_element_type=jnp.float32)
    h3 = jax.lax.dot_general(x, w3t, dn, preferred_element_type=jnp.float32)
    g = h1 * jax.lax.logistic(h1) * h3
    y = jax.lax.dot_general(g, w2t, dn, preferred_element_type=jnp.float32)

    @pl.when((j == 0) & (e == 0))
    def _():
        out_ref[...] = jnp.zeros_like(out_ref)

    sel = sel_ref[0]
    out_ref[...] += jax.lax.dot_general(
        sel, y, (((1,), (0,)), ((), ())), preferred_element_type=jnp.float32)


def kernel(x, expert_indices, w1, w2, w3):
    E, I, D = w1.shape
    T, K = expert_indices.shape
    TK = T * K
    IT = 2048
    J = I // IT

    idx_flat = expert_indices.astype(jnp.int32).reshape(-1)
    flat = jnp.sort(idx_flat)
    is_new = jnp.concatenate(
        [jnp.ones((1,), jnp.bool_), flat[1:] != flat[:-1]])
    pos = jnp.cumsum(is_new) - 1
    slots = jnp.full((E,), flat[-1], jnp.int32)
    slots = slots.at[pos].set(flat)
    valid = jnp.concatenate(
        [jnp.ones((1,), jnp.bool_), slots[1:] != slots[:-1]])
    assign = valid[:, None] & (idx_flat[None, :] == slots[:, None])
    onehot = (jnp.arange(TK)[:, None] // K
              == jnp.arange(T)[None, :]).astype(jnp.float32)
    sel = assign[:, :, None].astype(jnp.float32) * onehot[None]

    grid_spec = pltpu.PrefetchScalarGridSpec(
        num_scalar_prefetch=1,
        grid=(J, E),
        in_specs=[
            pl.BlockSpec((T, D), lambda j, e, slots: (0, 0)),
            pl.BlockSpec((1, TK, T), lambda j, e, slots: (e, 0, 0)),
            pl.BlockSpec((1, IT, D), lambda j, e, slots: (slots[e], j, 0)),
            pl.BlockSpec((1, IT, D), lambda j, e, slots: (slots[e], j, 0)),
            pl.BlockSpec((1, D, IT), lambda j, e, slots: (slots[e], 0, j)),
        ],
        out_specs=pl.BlockSpec((TK, D), lambda j, e, slots: (0, 0)),
    )

    out = pl.pallas_call(
        _ffn_body,
        grid_spec=grid_spec,
        out_shape=jax.ShapeDtypeStruct((TK, D), jnp.float32),
    )(slots, x, sel, w1, w3, w2)
    return out.reshape(T, K, D)


# speedup vs baseline: 9.4344x; 1.1497x over previous
import jax
import jax.numpy as jnp
from jax.experimental import pallas as pl
from jax.experimental.pallas import tpu as pltpu


def _ffn_body(slots_ref, valids_ref, x_ref, sel_ref, w1_ref, w3_ref, w2_ref,
              out_ref):
    j = pl.program_id(0)
    e = pl.program_id(1)

    @pl.when((j == 0) & (e == 0))
    def _():
        out_ref[...] = jnp.zeros_like(out_ref)

    @pl.when(valids_ref[e] == 1)
    def _():
        x = x_ref[...]
        w1t = w1_ref[0]
        w3t = w3_ref[0]
        w2t = w2_ref[0]

        dn = (((1,), (1,)), ((), ()))
        h1 = jax.lax.dot_general(x, w1t, dn,
                                 preferred_element_type=jnp.float32)
        h3 = jax.lax.dot_general(x, w3t, dn,
                                 preferred_element_type=jnp.float32)
        g = h1 * jax.lax.logistic(h1) * h3
        y = jax.lax.dot_general(g, w2t, dn,
                                preferred_element_type=jnp.float32)

        sel = sel_ref[0]
        out_ref[...] += jax.lax.dot_general(
            sel, y, (((1,), (0,)), ((), ())),
            preferred_element_type=jnp.float32)


def kernel(x, expert_indices, w1, w2, w3):
    E, I, D = w1.shape
    T, K = expert_indices.shape
    TK = T * K
    IT = 2048
    J = I // IT

    idx_flat = expert_indices.astype(jnp.int32).reshape(-1)
    flat = jnp.sort(idx_flat)
    is_new = jnp.concatenate(
        [jnp.ones((1,), jnp.bool_), flat[1:] != flat[:-1]])
    pos = jnp.cumsum(is_new) - 1
    slots = jnp.full((E,), flat[-1], jnp.int32)
    slots = slots.at[pos].set(flat)
    valid = jnp.concatenate(
        [jnp.ones((1,), jnp.bool_), slots[1:] != slots[:-1]])
    assign = valid[:, None] & (idx_flat[None, :] == slots[:, None])
    onehot = (jnp.arange(TK)[:, None] // K
              == jnp.arange(T)[None, :]).astype(jnp.float32)
    sel = assign[:, :, None].astype(jnp.float32) * onehot[None]

    grid_spec = pltpu.PrefetchScalarGridSpec(
        num_scalar_prefetch=2,
        grid=(J, E),
        in_specs=[
            pl.BlockSpec((T, D), lambda j, e, s, v: (0, 0)),
            pl.BlockSpec((1, TK, T), lambda j, e, s, v: (e, 0, 0)),
            pl.BlockSpec((1, IT, D), lambda j, e, s, v: (s[e], j, 0)),
            pl.BlockSpec((1, IT, D), lambda j, e, s, v: (s[e], j, 0)),
            pl.BlockSpec((1, D, IT), lambda j, e, s, v: (s[e], 0, j)),
        ],
        out_specs=pl.BlockSpec((TK, D), lambda j, e, s, v: (0, 0)),
    )

    out = pl.pallas_call(
        _ffn_body,
        grid_spec=grid_spec,
        out_shape=jax.ShapeDtypeStruct((TK, D), jnp.float32),
    )(slots, valid.astype(jnp.int32), x, sel, w1, w3, w2)
    return out.reshape(T, K, D)
